# Initial kernel scaffold; baseline (speedup 1.0000x reference)
import jax
import jax.numpy as jnp
from jax.experimental import pallas as pl


def kernel(x_in, sigma, alpha, f_w, f_b, p_w, p_b, mlp_w1, mlp_b1, mlp_w2, mlp_b2):
    raise NotImplementedError("write your pallas kernel here")


# speedup vs baseline: 5.8980x; 5.8980x over previous
import jax
import jax.numpy as jnp
from jax.experimental import pallas as pl
from jax.experimental.pallas import tpu as pltpu

_WS = 7
_HP = 16
_WP = 16
_N = _HP * _WP
_K = 9
_ENC = -4.0


def _sds_kernel(par_ref, out_ref):
    row = jax.lax.broadcasted_iota(jnp.int32, (_N, _N), 0)
    col = jax.lax.broadcasted_iota(jnp.int32, (_N, _N), 1)
    di = row // _WP - col // _WP
    dj = row % _WP - col % _WP
    d2 = (di * di + dj * dj).astype(jnp.float32)
    dist = jnp.sqrt(d2)
    sigma = par_ref[0:1, 0:1]
    alpha = par_ref[0:1, 1:2]
    out_ref[...] = (1.0 - alpha) * jnp.exp(-(dist * dist) / (2.0 * sigma * sigma))


def _patch_kernel(x_ref, fwT_ref, fb_ref, pwT_ref, pb_ref, par_ref, sds_ref,
                  out_ref):
    x = x_ref[0]
    fmat = jnp.dot(x, fwT_ref[...], preferred_element_type=jnp.float32) \
        + fb_ref[...]

    nrm = jnp.sqrt(jnp.sum(fmat * fmat, axis=1, keepdims=True))
    xn = fmat / jnp.maximum(nrm, 1e-8)
    alpha = par_ref[0:1, 1:2]
    sigma = par_ref[0:1, 0:1]

    cur = jax.lax.dot_general(
        xn * alpha, xn, dimension_numbers=(((1,), (1,)), ((), ())),
        preferred_element_type=jnp.float32) + sds_ref[...]

    col = jax.lax.broadcasted_iota(jnp.int32, (_N, _N), 1)
    rowv = jax.lax.broadcasted_iota(jnp.int32, (_N, 1), 0)

    wsum = jnp.zeros((_N, 1), jnp.float32)
    for _ in range(_K):
        m = jnp.max(cur, axis=1, keepdims=True)
        cand = jnp.where(cur >= m, col, _N)
        pos = jnp.min(cand, axis=1, keepdims=True)

        di = rowv // _WP - pos // _WP
        dj = rowv % _WP - pos % _WP
        d2 = (di * di + dj * dj).astype(jnp.float32)
        dist = jnp.sqrt(d2)
        sd = jnp.exp(-(dist * dist) / (2.0 * sigma * sigma))
        sf = (m - (1.0 - alpha) * sd) / alpha

        acc = par_ref[0:1, 15:16]
        for u in range(4):
            h = (sf * par_ref[0:1, 2 + 2 * u:3 + 2 * u]
                 + sd * par_ref[0:1, 3 + 2 * u:4 + 2 * u]
                 + par_ref[0:1, 10 + u:11 + u])
            h = h * jax.nn.sigmoid(h)
            acc = acc + h * par_ref[0:1, 16 + u:17 + u]
        we = jax.nn.sigmoid(acc)

        wsum = wsum + we
        cur = jnp.where(col == pos, we + _ENC, cur)

    inv = 1.0 / (wsum + 1e-12)
    adj = jnp.where(cur < _ENC * 0.5, (cur - _ENC) * inv, 0.0)

    agg = jnp.dot(adj, fmat, preferred_element_type=jnp.float32)
    outc = jnp.dot(agg, pwT_ref[...], preferred_element_type=jnp.float32)
    out_ref[0] = outc + pb_ref[...]


def kernel(x_in, sigma, alpha, f_w, f_b, p_w, p_b, mlp_w1, mlp_b1, mlp_w2, mlp_b2):
    B, C, H, W = x_in.shape
    d4 = f_w.shape[0]
    Bp = _WS * _WS

    xr = x_in.reshape(C, _WS, _HP, _WS, _WP).transpose(1, 3, 2, 4, 0)
    xr = xr.reshape(Bp, _N, C)

    fwT = f_w.T
    pwT = p_w.T
    fb = f_b.reshape(1, d4)
    pb = p_b.reshape(1, C)

    params = jnp.concatenate([
        sigma.reshape(1), alpha.reshape(1),
        mlp_w1.reshape(-1), mlp_b1.reshape(-1),
        jnp.zeros((1,), jnp.float32),
        mlp_b2.reshape(-1), mlp_w2.reshape(-1),
        jnp.zeros((12,), jnp.float32),
    ]).reshape(1, 32)

    sds = pl.pallas_call(
        _sds_kernel,
        out_shape=jax.ShapeDtypeStruct((_N, _N), jnp.float32),
    )(params)

    out_nodes = pl.pallas_call(
        _patch_kernel,
        grid=(Bp,),
        in_specs=[
            pl.BlockSpec((1, _N, C), lambda b: (b, 0, 0)),
            pl.BlockSpec((C, d4), lambda b: (0, 0)),
            pl.BlockSpec((1, d4), lambda b: (0, 0)),
            pl.BlockSpec((d4, C), lambda b: (0, 0)),
            pl.BlockSpec((1, C), lambda b: (0, 0)),
            pl.BlockSpec((1, 32), lambda b: (0, 0)),
            pl.BlockSpec((_N, _N), lambda b: (0, 0)),
        ],
        out_specs=pl.BlockSpec((1, _N, C), lambda b: (b, 0, 0)),
        out_shape=jax.ShapeDtypeStruct((Bp, _N, C), jnp.float32),
        compiler_params=pltpu.CompilerParams(
            dimension_semantics=("parallel",),
        ),
    )(xr, fwT, fb, pwT, pb, params, sds)

    out = out_nodes.reshape(_WS, _WS, _HP, _WP, C).transpose(4, 0, 2, 1, 3)
    return out.reshape(1, C, H * W)
